# baseline (speedup 1.0000x reference)
import functools

import jax
import jax.numpy as jnp
from jax import lax
from jax.experimental import pallas as pl
from jax.experimental.pallas import tpu as pltpu
from jax.experimental.pallas import tpu_sc as plsc

VOCAB = 32
EMBED = 16
N_TOKENS = 4 * 8192

_info = plsc.get_sparse_core_info()
_NC, _NS, _L = 1, _info.num_subcores, _info.num_lanes
_NW = _NC * _NS
_SC_ROWS = 2
_SC_TOKENS = _SC_ROWS * 8192
_TPW = _SC_TOKENS // _NW

_sc_mesh = plsc.VectorSubcoreMesh(
    core_axis_name="c", subcore_axis_name="s", num_cores=1
)


@functools.partial(
    pl.kernel,
    mesh=_sc_mesh,
    compiler_params=pltpu.CompilerParams(needs_layout_passes=False),
    out_type=jax.ShapeDtypeStruct((_NW, VOCAB, VOCAB), jnp.float32),
    scratch_types=[
        pltpu.VMEM((_TPW,), jnp.int32),
        pltpu.VMEM((_TPW,), jnp.int32),
        pltpu.VMEM((VOCAB, VOCAB), jnp.float32),
        pltpu.SemaphoreType.DMA,
        pltpu.SemaphoreType.DMA,
    ],
)
def _sc_pair_hist(idx_hbm, tgt_hbm, out_hbm, idx_v, tgt_v, hist_v, sem1, sem2):
    wid = lax.axis_index("s") * _NC + lax.axis_index("c")
    row = wid // (8192 // _TPW)
    col = (wid % (8192 // _TPW)) * _TPW
    cp_i = pltpu.async_copy(idx_hbm.at[row, pl.ds(col, _TPW)], idx_v, sem1)
    cp_t = pltpu.async_copy(tgt_hbm.at[row, pl.ds(col, _TPW)], tgt_v, sem2)

    zeros = jnp.zeros((_L,), jnp.float32)

    @plsc.parallel_loop(0, 2 * VOCAB, unroll=4)
    def _zero(i):
        r = i // 2
        c = (i % 2) * _L
        hist_v[r, pl.ds(c, _L)] = zeros

    cp_i.wait()
    cp_t.wait()

    ones = jnp.ones((_L,), jnp.float32)

    @plsc.parallel_loop(0, _TPW // _L, unroll=8)
    def _scatter(i):
        s = i * _L
        iv = idx_v[pl.ds(s, _L)]
        tv = tgt_v[pl.ds(s, _L)]
        plsc.addupdate_scatter(hist_v, [iv, tv], ones)
    pltpu.sync_copy(hist_v, out_hbm.at[wid])


def _tc_hist_body(idx_ref, tgt_ref, out_ref):
    iv = jax.lax.broadcasted_iota(jnp.int32, (VOCAB, 8192), 0)
    acc = jnp.zeros((VOCAB, VOCAB), jnp.float32)
    for r in range(_SC_ROWS, 4):
        row_i = idx_ref[pl.ds(r, 1), :]
        row_t = tgt_ref[pl.ds(r, 1), :]
        a = (iv == row_i).astype(jnp.bfloat16)
        bm = (iv == row_t).astype(jnp.bfloat16)
        acc += jax.lax.dot_general(
            a, bm, (((1,), (1,)), ((), ())),
            preferred_element_type=jnp.float32,
        )
    out_ref[...] = acc


def _tc_pair_hist(idx, targets):
    return pl.pallas_call(
        _tc_hist_body,
        out_shape=jax.ShapeDtypeStruct((VOCAB, VOCAB), jnp.float32),
    )(idx, targets)


def _combine_body(hist_ref, tc_hist_ref, embed_ref, w_ref, b_ref, out_ref):
    hsum = jnp.sum(hist_ref[...], axis=0) + tc_hist_ref[...]
    table = (
        jnp.dot(embed_ref[...], w_ref[...], preferred_element_type=jnp.float32)
        + b_ref[...]
    )
    m = jnp.max(table, axis=1, keepdims=True)
    lse = m + jnp.log(jnp.sum(jnp.exp(table - m), axis=1, keepdims=True))
    nll = lse - table
    loss = jnp.sum(hsum * nll, keepdims=True) * (1.0 / N_TOKENS)
    out_ref[...] = loss.reshape(1, 1)


def _combine(hist, tc_hist, embed, W, b):
    return pl.pallas_call(
        _combine_body,
        out_shape=jax.ShapeDtypeStruct((1, 1), jnp.float32),
    )(hist, tc_hist, embed, W, b.reshape(1, VOCAB))


def kernel(idx, targets, embed, W, b):
    tc_hist = _tc_pair_hist(idx, targets)
    sc_hist = _sc_pair_hist(idx, targets)
    return _combine(sc_hist, tc_hist, embed, W, b).reshape(())


# speedup vs baseline: 1.0083x; 1.0083x over previous
import functools

import jax
import jax.numpy as jnp
from jax import lax
from jax.experimental import pallas as pl
from jax.experimental.pallas import tpu as pltpu
from jax.experimental.pallas import tpu_sc as plsc

VOCAB = 32
EMBED = 16
N_TOKENS = 4 * 8192

_info = plsc.get_sparse_core_info()
_NC, _NS, _L = 1, _info.num_subcores, _info.num_lanes
_NW = _NC * _NS
_TPW = N_TOKENS // _NW

_sc_mesh = plsc.VectorSubcoreMesh(
    core_axis_name="c", subcore_axis_name="s", num_cores=1
)


@functools.partial(
    pl.kernel,
    mesh=_sc_mesh,
    compiler_params=pltpu.CompilerParams(needs_layout_passes=False),
    out_type=jax.ShapeDtypeStruct((_NW, VOCAB, VOCAB), jnp.float32),
    scratch_types=[
        pltpu.VMEM((_TPW,), jnp.int32),
        pltpu.VMEM((_TPW,), jnp.int32),
        pltpu.VMEM((VOCAB, VOCAB), jnp.float32),
        pltpu.SemaphoreType.DMA,
        pltpu.SemaphoreType.DMA,
    ],
)
def _sc_pair_hist(idx_hbm, tgt_hbm, out_hbm, idx_v, tgt_v, hist_v, sem1, sem2):
    wid = lax.axis_index("s") * _NC + lax.axis_index("c")
    row = wid // (8192 // _TPW)
    col = (wid % (8192 // _TPW)) * _TPW
    cp_i = pltpu.async_copy(idx_hbm.at[row, pl.ds(col, _TPW)], idx_v, sem1)
    cp_t = pltpu.async_copy(tgt_hbm.at[row, pl.ds(col, _TPW)], tgt_v, sem2)

    zeros = jnp.zeros((_L,), jnp.float32)

    @plsc.parallel_loop(0, 2 * VOCAB, unroll=4)
    def _zero(i):
        r = i // 2
        c = (i % 2) * _L
        hist_v[r, pl.ds(c, _L)] = zeros

    cp_i.wait()
    cp_t.wait()

    ones = jnp.ones((_L,), jnp.float32)

    @plsc.parallel_loop(0, _TPW // _L, unroll=8)
    def _scatter(i):
        s = i * _L
        iv = idx_v[pl.ds(s, _L)]
        tv = tgt_v[pl.ds(s, _L)]
        plsc.addupdate_scatter(hist_v, [iv, tv], ones)
    pltpu.sync_copy(hist_v, out_hbm.at[wid])


def _combine_body(hist_ref, embed_ref, w_ref, b_ref, out_ref):
    hsum = jnp.sum(hist_ref[...], axis=0)
    table = (
        jnp.dot(embed_ref[...], w_ref[...], preferred_element_type=jnp.float32)
        + b_ref[...]
    )
    m = jnp.max(table, axis=1, keepdims=True)
    lse = m + jnp.log(jnp.sum(jnp.exp(table - m), axis=1, keepdims=True))
    nll = lse - table
    loss = jnp.sum(hsum * nll, keepdims=True) * (1.0 / N_TOKENS)
    out_ref[...] = loss.reshape(1, 1)


def _combine(hist, embed, W, b):
    return pl.pallas_call(
        _combine_body,
        out_shape=jax.ShapeDtypeStruct((1, 1), jnp.float32),
    )(hist, embed, W, b.reshape(1, VOCAB))


def kernel(idx, targets, embed, W, b):
    hist = _sc_pair_hist(idx, targets)
    return _combine(hist, embed, W, b).reshape(())
